# baseline (speedup 1.0000x reference)
import jax
import jax.numpy as jnp
from jax import lax
from jax.experimental import pallas as pl
from jax.experimental.pallas import tpu as pltpu
from jax.experimental.pallas import tpu_sc as plsc

_U = 50000
_I = 50000
_UP = 50048
_D = 64
_H = 32
_E = 800000
_EP = 819200
_NCHUNK = _EP // 128
_CB = 8
_B = 16384
_NC = 2
_NS = 16
_RPS = _UP // _NS
_ZC = 136

_mesh = plsc.VectorSubcoreMesh(
    core_axis_name="c", subcore_axis_name="s", num_cores=_NC, num_subcores=_NS)


def _bcast_lane(vec, lane):
    idx = jnp.full((16, 1), lane, jnp.int32)
    return lax.gather(
        vec, idx,
        lax.GatherDimensionNumbers(
            offset_dims=(), collapsed_slice_dims=(0,), start_index_map=(0,)),
        (1,), mode=lax.GatherScatterMode.PROMISE_IN_BOUNDS)


_UC = 2
_NU = _NCHUNK // _NS // _UC


def _spmm_body(pk_hbm, vals_hbm, x_hbm, out_hbm,
               pk0, pk1, vl0, vl1, gi0, gi1, rw0, rw1, da0, da1,
               acc_sh, ps0, ps1, gs0, gs1, ss0, ss1):
    c = lax.axis_index("c")
    s = lax.axis_index("s")
    coff = c * _UP
    pk = (pk0, pk1)
    vl = (vl0, vl1)
    gi = (gi0, gi1)
    rw = (rw0, rw1)
    da = (da0, da1)
    ps = (ps0, ps1)
    gs = (gs0, gs1)
    ss = (ss0, ss1)
    cbase = s * (_NCHUNK // _NS)

    def zbody(i, _):
        z = jnp.zeros((16,), jnp.float32)
        da0[i, pl.ds(0, 16)] = z
        da0[i, pl.ds(16, 16)] = z
        return 0
    lax.fori_loop(0, _ZC, zbody, 0)
    def zcopy(k, _):
        pltpu.sync_copy(da0.at[pl.ds(0, _ZC)],
                        acc_sh.at[pl.ds(s * _RPS + k * _ZC, _ZC)])
        return 0
    lax.fori_loop(0, _RPS // _ZC, zcopy, 0)
    plsc.subcore_barrier()

    def unpack(b):
        for j in range(_UC):
            for l in range(8):
                sl = pl.ds(l * 16, 16)
                gi[b][j, sl] = pk[b][j, 1, sl] + coff
            for l in range(8):
                sl = pl.ds(l * 16, 16)
                rw[b][j, sl] = pk[b][j, 0, sl]

    def fire_gathers(b):
        for j in range(_UC):
            pltpu.async_copy(x_hbm.at[gi[b].at[j]],
                             da[b].at[pl.ds(j * 128, 128)], gs[b])

    def wait_gathers(b):
        pltpu.make_async_copy(x_hbm.at[pl.ds(0, _UC * 128)], da[b],
                              gs[b]).wait()

    def fire_scatters(b):
        for j in range(_UC):
            pltpu.async_copy(da[b].at[pl.ds(j * 128, 128)],
                             acc_sh.at[rw[b].at[j]], ss[b], add=True)

    def wait_scatters(b):
        pltpu.make_async_copy(da[b], acc_sh.at[pl.ds(0, _UC * 128)],
                              ss[b]).wait()

    def scale(b):
        for j in range(_UC):
            def sbody(g, _, b=b, j=j):
                vv = vl[b][j, pl.ds(g * 16, 16)]
                e0 = j * 128 + g * 16
                for e in range(16):
                    bv = _bcast_lane(vv, e)
                    r = e0 + e
                    da[b][r, pl.ds(0, 16)] = da[b][r, pl.ds(0, 16)] * bv
                    da[b][r, pl.ds(16, 16)] = da[b][r, pl.ds(16, 16)] * bv
                return 0
            lax.fori_loop(0, 8, sbody, 0)

    pltpu.sync_copy(pk_hbm.at[pl.ds(cbase, _UC)], pk0)
    pltpu.sync_copy(vals_hbm.at[pl.ds(cbase, _UC)], vl0)
    unpack(0)
    fire_gathers(0)

    def it_body(t, _):
        for b in range(2):
            k = 2 * t + b
            nb = 1 - b
            @pl.when(k <= _NU - 2)
            def _(k=k, nb=nb):
                pltpu.async_copy(pk_hbm.at[pl.ds(cbase + (k + 1) * _UC, _UC)],
                                 pk[nb], ps[nb])
                pltpu.async_copy(vals_hbm.at[pl.ds(cbase + (k + 1) * _UC, _UC)],
                                 vl[nb], ps[nb])
            @pl.when(k >= 1)
            def _(nb=nb):
                wait_scatters(nb)
            @pl.when(k <= _NU - 2)
            def _(k=k, nb=nb):
                pltpu.make_async_copy(
                    pk_hbm.at[pl.ds(cbase + (k + 1) * _UC, _UC)],
                    pk[nb], ps[nb]).wait()
                pltpu.make_async_copy(
                    vals_hbm.at[pl.ds(cbase + (k + 1) * _UC, _UC)],
                    vl[nb], ps[nb]).wait()
                unpack(nb)
                fire_gathers(nb)
            wait_gathers(b)
            scale(b)
            fire_scatters(b)
        return 0
    lax.fori_loop(0, _NU // 2, it_body, 0)
    wait_scatters(1)

    plsc.subcore_barrier()
    pltpu.sync_copy(acc_sh.at[pl.ds(s * _RPS, _RPS)],
                    out_hbm.at[pl.ds(c * _UP + s * _RPS, _RPS)])


_spmm = pl.kernel(
    _spmm_body,
    out_type=jax.ShapeDtypeStruct((_NC * _UP, _H), jnp.float32),
    mesh=_mesh,
    compiler_params=pltpu.CompilerParams(use_tc_tiling_on_sc=False),
    scratch_types=[
        pltpu.VMEM((_UC, 2, 128), jnp.int32),
        pltpu.VMEM((_UC, 2, 128), jnp.int32),
        pltpu.VMEM((_UC, 128), jnp.float32),
        pltpu.VMEM((_UC, 128), jnp.float32),
        pltpu.VMEM((_UC, 128), jnp.int32),
        pltpu.VMEM((_UC, 128), jnp.int32),
        pltpu.VMEM((_UC, 128), jnp.int32),
        pltpu.VMEM((_UC, 128), jnp.int32),
        pltpu.VMEM((_UC * 128, _H), jnp.float32),
        pltpu.VMEM((_UC * 128, _H), jnp.float32),
        pltpu.VMEM_SHARED((_UP, _H), jnp.float32),
        pltpu.SemaphoreType.DMA,
        pltpu.SemaphoreType.DMA,
        pltpu.SemaphoreType.DMA,
        pltpu.SemaphoreType.DMA,
        pltpu.SemaphoreType.DMA,
        pltpu.SemaphoreType.DMA,
    ],
)


def _bprgather_body(uidx_hbm, iidx_hbm, jidx_hbm,
                    ub_hbm, g1u_hbm, g2u_hbm, g3u_hbm,
                    ib_hbm, g1i_hbm, g2i_hbm, g3i_hbm,
                    ug_hbm, pig_hbm, pjg_hbm,
                    idx_v, t0_v, t1_v, t2_v, t3_v, o0_v, o1_v, sem, osem):
    c = lax.axis_index("c")
    s = lax.axis_index("s")
    coff = c * _UP
    ob = (o0_v, o1_v)

    def one(src_idx_hbm, tabs, dst_hbm):
        pltpu.sync_copy(src_idx_hbm.at[pl.ds(s * 8, 8)], idx_v)
        for j in range(8):
            for l in range(8):
                sl = pl.ds(l * 16, 16)
                idx_v[j, sl] = idx_v[j, sl] + coff
        for j in range(8):
            cps = []
            for tab, buf in zip(tabs, (t0_v, t1_v, t2_v, t3_v)):
                cps.append(pltpu.async_copy(
                    tab.at[idx_v.at[j]], buf, sem))
            for cp in cps:
                cp.wait()
            o = ob[j % 2]
            @pl.when(j >= 2)
            def _(j=j, dst_hbm=dst_hbm, o=o):
                pltpu.make_async_copy(
                    o, dst_hbm.at[pl.ds(c * _B + (s * 8 + j - 2) * 128, 128)],
                    osem).wait()
            def rbody(r, _, o=o):
                for off in (0, 16):
                    sl = pl.ds(off, 16)
                    o[r, sl] = (t0_v[r, sl] + 0.5 * t1_v[r, sl]
                                + (1.0 / 3.0) * t2_v[r, sl]
                                + 0.25 * t3_v[r, sl])
                return 0
            lax.fori_loop(0, 128, rbody, 0)
            pltpu.async_copy(
                o, dst_hbm.at[pl.ds(c * _B + (s * 8 + j) * 128, 128)], osem)
        for j in (6, 7):
            pltpu.make_async_copy(
                ob[j % 2],
                dst_hbm.at[pl.ds(c * _B + (s * 8 + j) * 128, 128)],
                osem).wait()

    one(uidx_hbm, (ub_hbm, g1u_hbm, g2u_hbm, g3u_hbm), ug_hbm)
    one(iidx_hbm, (ib_hbm, g1i_hbm, g2i_hbm, g3i_hbm), pig_hbm)
    one(jidx_hbm, (ib_hbm, g1i_hbm, g2i_hbm, g3i_hbm), pjg_hbm)


_bprgather = pl.kernel(
    _bprgather_body,
    out_type=(
        jax.ShapeDtypeStruct((_NC * _B, _H), jnp.float32),
        jax.ShapeDtypeStruct((_NC * _B, _H), jnp.float32),
        jax.ShapeDtypeStruct((_NC * _B, _H), jnp.float32),
    ),
    mesh=_mesh,
    compiler_params=pltpu.CompilerParams(use_tc_tiling_on_sc=False),
    scratch_types=[
        pltpu.VMEM((8, 128), jnp.int32),
        pltpu.VMEM((128, _H), jnp.float32),
        pltpu.VMEM((128, _H), jnp.float32),
        pltpu.VMEM((128, _H), jnp.float32),
        pltpu.VMEM((128, _H), jnp.float32),
        pltpu.VMEM((128, _H), jnp.float32),
        pltpu.VMEM((128, _H), jnp.float32),
        pltpu.SemaphoreType.DMA,
        pltpu.SemaphoreType.DMA,
    ],
)


def _bpr_tc(u0_ref, u1_ref, pi0_ref, pi1_ref, pj0_ref, pj1_ref, out_ref):
    i = pl.program_id(0)

    def dots(a_ref, b_ref):
        p = a_ref[...] * b_ref[...]
        return jnp.sum(p.reshape(128, 4, _H), axis=2)

    x = (dots(u0_ref, pi0_ref) + dots(u1_ref, pi1_ref)
         - dots(u0_ref, pj0_ref) - dots(u1_ref, pj1_ref))
    sp = jnp.maximum(-x, 0.0) + jnp.log1p(jnp.exp(-jnp.abs(x)))
    reg = (jnp.sum(u0_ref[...] ** 2 + pi0_ref[...] ** 2 + pj0_ref[...] ** 2)
           + jnp.sum(u1_ref[...] ** 2 + pi1_ref[...] ** 2
                     + pj1_ref[...] ** 2))
    val = jnp.sum(sp) / _B + 1e-4 * reg / _B

    @pl.when(i == 0)
    def _():
        out_ref[...] = jnp.zeros_like(out_ref)
    out_ref[...] += val


_SR = 136


def _self_tc(b0_ref, b1_ref, g10_ref, g11_ref, g20_ref, g21_ref,
             g30_ref, g31_ref, o0_ref, o1_ref, nu_ref,
             c0_ref, c1_ref, h10_ref, h11_ref, h20_ref, h21_ref,
             h30_ref, h31_ref, p0_ref, p1_ref, ni_ref, out_ref):
    i = pl.program_id(0)

    def ssq_part(b, g1, g2, g3, o):
        g = (b[...] + 0.5 * g1[...] + (1.0 / 3.0) * g2[...]
             + 0.25 * g3[...])
        d = o[...] - g
        return jnp.sum((d * d).reshape(_SR, 4, _H), axis=2)

    rowid = (jax.lax.broadcasted_iota(jnp.int32, (_SR, 4), 0)
             + i * _SR) * 4 + jax.lax.broadcasted_iota(jnp.int32, (_SR, 4), 1)
    valid = rowid < _U

    def side(b0, b1, g10, g11, g20, g21, g30, g31, o0, o1, n):
        ssq = (ssq_part(b0, g10, g20, g30, o0)
               + ssq_part(b1, g11, g21, g31, o1))
        du = jnp.sqrt(jnp.where(valid, ssq, 0.0))
        return jnp.sum(jnp.where(valid, du * n[...], 0.0))

    val = (side(b0_ref, b1_ref, g10_ref, g11_ref, g20_ref, g21_ref,
                g30_ref, g31_ref, o0_ref, o1_ref, nu_ref) / _U
           + side(c0_ref, c1_ref, h10_ref, h11_ref, h20_ref, h21_ref,
                  h30_ref, h31_ref, p0_ref, p1_ref, ni_ref) / _I)

    @pl.when(i == 0)
    def _():
        out_ref[...] = jnp.zeros_like(out_ref)
    out_ref[...] += val


def kernel(user, item_i, item_j, edge_u, edge_i, edge_vals,
           embed_user, embed_item, old_U_emb, old_I_emb, n_U, n_I):
    f32 = jnp.float32
    zrow = jnp.zeros((_UP - _U, _H), f32)
    ue2 = jnp.concatenate(
        [embed_user[:, :_H], zrow, embed_user[:, _H:], zrow], axis=0)
    ie2 = jnp.concatenate(
        [embed_item[:, :_H], zrow, embed_item[:, _H:], zrow], axis=0)
    pad = _EP - _E
    zpad_i = jnp.zeros((pad,), jnp.int32)
    rows_u = jnp.concatenate([edge_u.astype(jnp.int32), zpad_i]).reshape(_NCHUNK, 128)
    rows_i = jnp.concatenate([edge_i.astype(jnp.int32), zpad_i]).reshape(_NCHUNK, 128)
    vals2 = jnp.concatenate(
        [edge_vals.astype(f32), jnp.zeros((pad,), f32)]).reshape(_NCHUNK, 128)
    pk_ud = jnp.stack([rows_u, rows_i], axis=1)
    pk_id = jnp.stack([rows_i, rows_u], axis=1)

    g1u = _spmm(pk_ud, vals2, ie2)
    g1i = _spmm(pk_id, vals2, ue2)
    g2u = _spmm(pk_ud, vals2, g1i)
    g2i = _spmm(pk_id, vals2, g1u)
    g3u = _spmm(pk_ud, vals2, g2i)
    g3i = _spmm(pk_id, vals2, g2u)

    u2d = user.astype(jnp.int32).reshape(128, 128)
    i2d = item_i.astype(jnp.int32).reshape(128, 128)
    j2d = item_j.astype(jnp.int32).reshape(128, 128)
    ug, pig, pjg = _bprgather(
        u2d, i2d, j2d, ue2, g1u, g2u, g3u, ie2, g1i, g2i, g3i)

    def b128(t):
        return t.reshape(_NC * _B // 4, 128)

    q0 = pl.BlockSpec((128, 128), lambda i: (i, 0))
    q1 = pl.BlockSpec((128, 128), lambda i: (i + 32, 0))
    bpr = pl.pallas_call(
        _bpr_tc,
        grid=(32,),
        in_specs=[q0, q1, q0, q1, q0, q1],
        out_specs=pl.BlockSpec((1, 1), lambda i: (0, 0)),
        out_shape=jax.ShapeDtypeStruct((1, 1), f32),
    )(b128(ug), b128(ug), b128(pig), b128(pig), b128(pjg), b128(pjg))

    npad = jnp.zeros((_UP - _U,), f32)
    old_u_s = jnp.concatenate(
        [old_U_emb[:, :_H], zrow, old_U_emb[:, _H:], zrow], axis=0)
    old_i_s = jnp.concatenate(
        [old_I_emb[:, :_H], zrow, old_I_emb[:, _H:], zrow], axis=0)
    n_u_p = jnp.concatenate([n_U.astype(f32), npad]).reshape(_UP // 4, 4)
    n_i_p = jnp.concatenate([n_I.astype(f32), npad]).reshape(_UP // 4, 4)

    def v128(t):
        return t.reshape(_NC * _UP // 4, 128)

    h0 = pl.BlockSpec((_SR, 128), lambda i: (i, 0))
    h1 = pl.BlockSpec((_SR, 128), lambda i: (i + 92, 0))
    nspec = pl.BlockSpec((_SR, 4), lambda i: (i, 0))
    selfv = pl.pallas_call(
        _self_tc,
        grid=(92,),
        in_specs=[h0, h1, h0, h1, h0, h1, h0, h1, h0, h1, nspec,
                  h0, h1, h0, h1, h0, h1, h0, h1, h0, h1, nspec],
        out_specs=pl.BlockSpec((1, 1), lambda i: (0, 0)),
        out_shape=jax.ShapeDtypeStruct((1, 1), f32),
    )(v128(ue2), v128(ue2), v128(g1u), v128(g1u), v128(g2u), v128(g2u),
      v128(g3u), v128(g3u), v128(old_u_s), v128(old_u_s), n_u_p,
      v128(ie2), v128(ie2), v128(g1i), v128(g1i), v128(g2i), v128(g2i),
      v128(g3i), v128(g3i), v128(old_i_s), v128(old_i_s), n_i_p)

    loss_bpr = bpr[0, 0]
    loss_self = selfv[0, 0]
    one = jnp.array(1.0, dtype=f32)
    return (loss_bpr, 100.0 * loss_self, one, one)


# speedup vs baseline: 1.0208x; 1.0208x over previous
import jax
import jax.numpy as jnp
from jax import lax
from jax.experimental import pallas as pl
from jax.experimental.pallas import tpu as pltpu
from jax.experimental.pallas import tpu_sc as plsc

_U = 50000
_I = 50000
_UP = 50048
_D = 64
_H = 32
_E = 800000
_EP = 819200
_NCHUNK = _EP // 128
_CB = 8
_B = 16384
_NC = 2
_NS = 16
_RPS = _UP // _NS
_ZC = 136

_mesh = plsc.VectorSubcoreMesh(
    core_axis_name="c", subcore_axis_name="s", num_cores=_NC, num_subcores=_NS)


def _bcast_lane(vec, lane):
    idx = jnp.full((16, 1), lane, jnp.int32)
    return lax.gather(
        vec, idx,
        lax.GatherDimensionNumbers(
            offset_dims=(), collapsed_slice_dims=(0,), start_index_map=(0,)),
        (1,), mode=lax.GatherScatterMode.PROMISE_IN_BOUNDS)


_UC = 2
_NU = _NCHUNK // _NS // _UC


def _spmm_body(pk_hbm, vals_hbm, x_hbm, out_hbm,
               pk0, pk1, vl0, vl1, gi0, gi1, rw0, rw1, da0, da1,
               acc_sh, ps0, ps1, gs0, gs1, ss0, ss1):
    c = lax.axis_index("c")
    s = lax.axis_index("s")
    coff = c * _UP
    pk = (pk0, pk1)
    vl = (vl0, vl1)
    gi = (gi0, gi1)
    rw = (rw0, rw1)
    da = (da0, da1)
    ps = (ps0, ps1)
    gs = (gs0, gs1)
    ss = (ss0, ss1)
    cbase = s * (_NCHUNK // _NS)

    def zbody(i, _):
        z = jnp.zeros((16,), jnp.float32)
        da0[i, pl.ds(0, 16)] = z
        da0[i, pl.ds(16, 16)] = z
        return 0
    lax.fori_loop(0, _ZC, zbody, 0)
    def zcopy(k, _):
        pltpu.sync_copy(da0.at[pl.ds(0, _ZC)],
                        acc_sh.at[pl.ds(s * _RPS + k * _ZC, _ZC)])
        return 0
    lax.fori_loop(0, _RPS // _ZC, zcopy, 0)
    plsc.subcore_barrier()

    def unpack(b):
        for j in range(_UC):
            for l in range(8):
                sl = pl.ds(l * 16, 16)
                gi[b][j, sl] = pk[b][j, 1, sl] + coff
            for l in range(8):
                sl = pl.ds(l * 16, 16)
                rw[b][j, sl] = pk[b][j, 0, sl]

    def fire_gathers(b):
        for j in range(_UC):
            pltpu.async_copy(x_hbm.at[gi[b].at[j]],
                             da[b].at[pl.ds(j * 128, 128)], gs[b])

    def wait_gathers(b):
        pltpu.make_async_copy(x_hbm.at[pl.ds(0, _UC * 128)], da[b],
                              gs[b]).wait()

    def fire_scatters(b):
        for j in range(_UC):
            pltpu.async_copy(da[b].at[pl.ds(j * 128, 128)],
                             acc_sh.at[rw[b].at[j]], ss[b], add=True)

    def wait_scatters(b):
        pltpu.make_async_copy(da[b], acc_sh.at[pl.ds(0, _UC * 128)],
                              ss[b]).wait()

    def scale(b):
        for j in range(_UC):
            def sbody(g, _, b=b, j=j):
                vv = vl[b][j, pl.ds(g * 16, 16)]
                e0 = j * 128 + g * 16
                for e in range(16):
                    bv = _bcast_lane(vv, e)
                    r = e0 + e
                    da[b][r, pl.ds(0, 16)] = da[b][r, pl.ds(0, 16)] * bv
                    da[b][r, pl.ds(16, 16)] = da[b][r, pl.ds(16, 16)] * bv
                return 0
            lax.fori_loop(0, 8, sbody, 0)

    pltpu.sync_copy(pk_hbm.at[pl.ds(cbase, _UC)], pk0)
    pltpu.sync_copy(vals_hbm.at[pl.ds(cbase, _UC)], vl0)
    unpack(0)
    fire_gathers(0)

    def it_body(t, _):
        for b in range(2):
            k = 2 * t + b
            nb = 1 - b
            @pl.when(k <= _NU - 2)
            def _(k=k, nb=nb):
                pltpu.async_copy(pk_hbm.at[pl.ds(cbase + (k + 1) * _UC, _UC)],
                                 pk[nb], ps[nb])
                pltpu.async_copy(vals_hbm.at[pl.ds(cbase + (k + 1) * _UC, _UC)],
                                 vl[nb], ps[nb])
            @pl.when(k >= 1)
            def _(nb=nb):
                wait_scatters(nb)
            @pl.when(k <= _NU - 2)
            def _(k=k, nb=nb):
                pltpu.make_async_copy(
                    pk_hbm.at[pl.ds(cbase + (k + 1) * _UC, _UC)],
                    pk[nb], ps[nb]).wait()
                pltpu.make_async_copy(
                    vals_hbm.at[pl.ds(cbase + (k + 1) * _UC, _UC)],
                    vl[nb], ps[nb]).wait()
                unpack(nb)
                fire_gathers(nb)
            wait_gathers(b)
            scale(b)
            fire_scatters(b)
        return 0
    lax.fori_loop(0, _NU // 2, it_body, 0)
    wait_scatters(1)

    plsc.subcore_barrier()
    pltpu.sync_copy(acc_sh.at[pl.ds(s * _RPS, _RPS)],
                    out_hbm.at[pl.ds(c * _UP + s * _RPS, _RPS)])


_spmm = pl.kernel(
    _spmm_body,
    out_type=jax.ShapeDtypeStruct((_NC * _UP, _H), jnp.float32),
    mesh=_mesh,
    compiler_params=pltpu.CompilerParams(use_tc_tiling_on_sc=False),
    scratch_types=[
        pltpu.VMEM((_UC, 2, 128), jnp.int32),
        pltpu.VMEM((_UC, 2, 128), jnp.int32),
        pltpu.VMEM((_UC, 128), jnp.float32),
        pltpu.VMEM((_UC, 128), jnp.float32),
        pltpu.VMEM((_UC, 128), jnp.int32),
        pltpu.VMEM((_UC, 128), jnp.int32),
        pltpu.VMEM((_UC, 128), jnp.int32),
        pltpu.VMEM((_UC, 128), jnp.int32),
        pltpu.VMEM((_UC * 128, _H), jnp.float32),
        pltpu.VMEM((_UC * 128, _H), jnp.float32),
        pltpu.VMEM_SHARED((_UP, _H), jnp.float32),
        pltpu.SemaphoreType.DMA,
        pltpu.SemaphoreType.DMA,
        pltpu.SemaphoreType.DMA,
        pltpu.SemaphoreType.DMA,
        pltpu.SemaphoreType.DMA,
        pltpu.SemaphoreType.DMA,
    ],
)


def _bprgather_body(uidx_hbm, iidx_hbm, jidx_hbm,
                    ub_hbm, g1u_hbm, g2u_hbm, g3u_hbm,
                    ib_hbm, g1i_hbm, g2i_hbm, g3i_hbm,
                    ug_hbm, pig_hbm, pjg_hbm,
                    idx_v, t0_v, t1_v, t2_v, t3_v, o0_v, o1_v, sem, osem):
    c = lax.axis_index("c")
    s = lax.axis_index("s")
    coff = c * _UP
    ob = (o0_v, o1_v)

    def one(src_idx_hbm, tabs, dst_hbm):
        pltpu.sync_copy(src_idx_hbm.at[pl.ds(s * 8, 8)], idx_v)
        for j in range(8):
            for l in range(8):
                sl = pl.ds(l * 16, 16)
                idx_v[j, sl] = idx_v[j, sl] + coff
        for j in range(8):
            cps = []
            for tab, buf in zip(tabs, (t0_v, t1_v, t2_v, t3_v)):
                cps.append(pltpu.async_copy(
                    tab.at[idx_v.at[j]], buf, sem))
            for cp in cps:
                cp.wait()
            o = ob[j % 2]
            @pl.when(j >= 2)
            def _(j=j, dst_hbm=dst_hbm, o=o):
                pltpu.make_async_copy(
                    o, dst_hbm.at[pl.ds(c * _B + (s * 8 + j - 2) * 128, 128)],
                    osem).wait()
            def rbody(r, _, o=o):
                for off in (0, 16):
                    sl = pl.ds(off, 16)
                    o[r, sl] = (t0_v[r, sl] + 0.5 * t1_v[r, sl]
                                + (1.0 / 3.0) * t2_v[r, sl]
                                + 0.25 * t3_v[r, sl])
                return 0
            lax.fori_loop(0, 128, rbody, 0)
            pltpu.async_copy(
                o, dst_hbm.at[pl.ds(c * _B + (s * 8 + j) * 128, 128)], osem)
        for j in (6, 7):
            pltpu.make_async_copy(
                ob[j % 2],
                dst_hbm.at[pl.ds(c * _B + (s * 8 + j) * 128, 128)],
                osem).wait()

    one(uidx_hbm, (ub_hbm, g1u_hbm, g2u_hbm, g3u_hbm), ug_hbm)
    one(iidx_hbm, (ib_hbm, g1i_hbm, g2i_hbm, g3i_hbm), pig_hbm)
    one(jidx_hbm, (ib_hbm, g1i_hbm, g2i_hbm, g3i_hbm), pjg_hbm)


_bprgather = pl.kernel(
    _bprgather_body,
    out_type=(
        jax.ShapeDtypeStruct((_NC * _B, _H), jnp.float32),
        jax.ShapeDtypeStruct((_NC * _B, _H), jnp.float32),
        jax.ShapeDtypeStruct((_NC * _B, _H), jnp.float32),
    ),
    mesh=_mesh,
    compiler_params=pltpu.CompilerParams(use_tc_tiling_on_sc=False),
    scratch_types=[
        pltpu.VMEM((8, 128), jnp.int32),
        pltpu.VMEM((128, _H), jnp.float32),
        pltpu.VMEM((128, _H), jnp.float32),
        pltpu.VMEM((128, _H), jnp.float32),
        pltpu.VMEM((128, _H), jnp.float32),
        pltpu.VMEM((128, _H), jnp.float32),
        pltpu.VMEM((128, _H), jnp.float32),
        pltpu.SemaphoreType.DMA,
        pltpu.SemaphoreType.DMA,
    ],
)


def _bpr_tc(u_ref, pi_ref, pj_ref, out_ref):
    i = pl.program_id(0)
    u = u_ref[...]
    pi = pi_ref[...]
    pj = pj_ref[...]
    x2 = jnp.sum(u * (pi - pj), axis=2)
    x = x2[0:1, :] + x2[1:2, :]
    sp = jnp.maximum(-x, 0.0) + jnp.log1p(jnp.exp(-jnp.abs(x)))
    reg = jnp.sum(u * u + pi * pi + pj * pj)
    val = jnp.sum(sp) / _B + 1e-4 * reg / _B

    @pl.when(i == 0)
    def _():
        out_ref[...] = jnp.zeros_like(out_ref)
    out_ref[...] += val


_SR = 136


def _self_tc(b0_ref, b1_ref, g10_ref, g11_ref, g20_ref, g21_ref,
             g30_ref, g31_ref, o0_ref, o1_ref, nu_ref,
             c0_ref, c1_ref, h10_ref, h11_ref, h20_ref, h21_ref,
             h30_ref, h31_ref, p0_ref, p1_ref, ni_ref, out_ref):
    i = pl.program_id(0)

    def ssq_part(b, g1, g2, g3, o):
        g = (b[...] + 0.5 * g1[...] + (1.0 / 3.0) * g2[...]
             + 0.25 * g3[...])
        d = o[...] - g
        return jnp.sum((d * d).reshape(_SR, 4, _H), axis=2)

    rowid = (jax.lax.broadcasted_iota(jnp.int32, (_SR, 4), 0)
             + i * _SR) * 4 + jax.lax.broadcasted_iota(jnp.int32, (_SR, 4), 1)
    valid = rowid < _U

    def side(b0, b1, g10, g11, g20, g21, g30, g31, o0, o1, n):
        ssq = (ssq_part(b0, g10, g20, g30, o0)
               + ssq_part(b1, g11, g21, g31, o1))
        du = jnp.sqrt(jnp.where(valid, ssq, 0.0))
        return jnp.sum(jnp.where(valid, du * n[...], 0.0))

    val = (side(b0_ref, b1_ref, g10_ref, g11_ref, g20_ref, g21_ref,
                g30_ref, g31_ref, o0_ref, o1_ref, nu_ref) / _U
           + side(c0_ref, c1_ref, h10_ref, h11_ref, h20_ref, h21_ref,
                  h30_ref, h31_ref, p0_ref, p1_ref, ni_ref) / _I)

    @pl.when(i == 0)
    def _():
        out_ref[...] = jnp.zeros_like(out_ref)
    out_ref[...] += val


def kernel(user, item_i, item_j, edge_u, edge_i, edge_vals,
           embed_user, embed_item, old_U_emb, old_I_emb, n_U, n_I):
    f32 = jnp.float32
    zrow = jnp.zeros((_UP - _U, _H), f32)
    ue2 = jnp.concatenate(
        [embed_user[:, :_H], zrow, embed_user[:, _H:], zrow], axis=0)
    ie2 = jnp.concatenate(
        [embed_item[:, :_H], zrow, embed_item[:, _H:], zrow], axis=0)
    pad = _EP - _E
    zpad_i = jnp.zeros((pad,), jnp.int32)
    rows_u = jnp.concatenate([edge_u.astype(jnp.int32), zpad_i]).reshape(_NCHUNK, 128)
    rows_i = jnp.concatenate([edge_i.astype(jnp.int32), zpad_i]).reshape(_NCHUNK, 128)
    vals2 = jnp.concatenate(
        [edge_vals.astype(f32), jnp.zeros((pad,), f32)]).reshape(_NCHUNK, 128)
    pk_ud = jnp.stack([rows_u, rows_i], axis=1)
    pk_id = jnp.stack([rows_i, rows_u], axis=1)

    g1u = _spmm(pk_ud, vals2, ie2)
    g1i = _spmm(pk_id, vals2, ue2)
    g2u = _spmm(pk_ud, vals2, g1i)
    g2i = _spmm(pk_id, vals2, g1u)
    g3u = _spmm(pk_ud, vals2, g2i)
    g3i = _spmm(pk_id, vals2, g2u)

    u2d = user.astype(jnp.int32).reshape(128, 128)
    i2d = item_i.astype(jnp.int32).reshape(128, 128)
    j2d = item_j.astype(jnp.int32).reshape(128, 128)
    ug, pig, pjg = _bprgather(
        u2d, i2d, j2d, ue2, g1u, g2u, g3u, ie2, g1i, g2i, g3i)

    bpr = pl.pallas_call(
        _bpr_tc,
        grid=(16,),
        in_specs=[pl.BlockSpec((2, 1024, _H), lambda i: (0, i, 0))] * 3,
        out_specs=pl.BlockSpec((1, 1), lambda i: (0, 0)),
        out_shape=jax.ShapeDtypeStruct((1, 1), f32),
    )(ug.reshape(2, _B, _H), pig.reshape(2, _B, _H), pjg.reshape(2, _B, _H))

    npad = jnp.zeros((_UP - _U,), f32)
    old_u_s = jnp.concatenate(
        [old_U_emb[:, :_H], zrow, old_U_emb[:, _H:], zrow], axis=0)
    old_i_s = jnp.concatenate(
        [old_I_emb[:, :_H], zrow, old_I_emb[:, _H:], zrow], axis=0)
    n_u_p = jnp.concatenate([n_U.astype(f32), npad]).reshape(_UP // 4, 4)
    n_i_p = jnp.concatenate([n_I.astype(f32), npad]).reshape(_UP // 4, 4)

    def v128(t):
        return t.reshape(_NC * _UP // 4, 128)

    h0 = pl.BlockSpec((_SR, 128), lambda i: (i, 0))
    h1 = pl.BlockSpec((_SR, 128), lambda i: (i + 92, 0))
    nspec = pl.BlockSpec((_SR, 4), lambda i: (i, 0))
    selfv = pl.pallas_call(
        _self_tc,
        grid=(92,),
        in_specs=[h0, h1, h0, h1, h0, h1, h0, h1, h0, h1, nspec,
                  h0, h1, h0, h1, h0, h1, h0, h1, h0, h1, nspec],
        out_specs=pl.BlockSpec((1, 1), lambda i: (0, 0)),
        out_shape=jax.ShapeDtypeStruct((1, 1), f32),
    )(v128(ue2), v128(ue2), v128(g1u), v128(g1u), v128(g2u), v128(g2u),
      v128(g3u), v128(g3u), v128(old_u_s), v128(old_u_s), n_u_p,
      v128(ie2), v128(ie2), v128(g1i), v128(g1i), v128(g2i), v128(g2i),
      v128(g3i), v128(g3i), v128(old_i_s), v128(old_i_s), n_i_p)

    loss_bpr = bpr[0, 0]
    loss_self = selfv[0, 0]
    one = jnp.array(1.0, dtype=f32)
    return (loss_bpr, 100.0 * loss_self, one, one)
